# baseline (speedup 1.0000x reference)
import jax
import jax.numpy as jnp
from jax.experimental import pallas as pl
from jax.experimental.pallas import tpu as pltpu

_LANE = 128
_POOL_BLOCK_BYTES = 8 * 1024 * 1024
_VMEM_CAP = 48 * 1024 * 1024


def _round_up(v, m):
    return -(-v // m) * m


def _gate_rows(sum_rows, max_rows, w1, w2, inv_hw, n):
    p = jnp.concatenate([sum_rows * inv_hw, max_rows], axis=0)
    h = jax.lax.dot_general(p, w1, (((1,), (1,)), ((), ())),
                            preferred_element_type=jnp.float32)
    h = jnp.maximum(h, 0.0)
    o = jax.lax.dot_general(h, w2, (((1,), (1,)), ((), ())),
                            preferred_element_type=jnp.float32)
    return jax.nn.sigmoid(o[:n, :] + o[n:, :])


def kernel(x_nchw, w1, w2):
    N, C, H, W = x_nchw.shape
    Cout = w2.shape[0]
    HW = H * W
    inv_hw = 1.0 / float(HW)
    itemsize = jnp.dtype(x_nchw.dtype).itemsize

    x = x_nchw.reshape(N, C, HW)

    ipb = 1
    for cand in (4, 2):
        if N % cand == 0 and cand * C * HW * itemsize <= _POOL_BLOCK_BYTES:
            ipb = cand
            break
    nsteps = N // ipb
    block_bytes = ipb * _round_up(C, 8) * _round_up(HW, _LANE) * itemsize
    vmem_limit = int(min(_VMEM_CAP, 2 * block_bytes + 8 * 1024 * 1024))

    if block_bytes <= _POOL_BLOCK_BYTES:
        def body2(x_ref, w1_ref, w2_ref, o_ref, s_rows, m_rows):
            k = pl.program_id(0)
            xv = x_ref[...]
            s_rows[k] = jnp.sum(xv, axis=2)
            m_rows[k] = jnp.max(xv, axis=2)

            @pl.when(k == nsteps - 1)
            def _fin():
                s_all = jnp.concatenate(
                    [s_rows[i] for i in range(nsteps)], axis=0)
                m_all = jnp.concatenate(
                    [m_rows[i] for i in range(nsteps)], axis=0)
                gate = _gate_rows(s_all, m_all,
                                  w1_ref[...].astype(jnp.float32),
                                  w2_ref[...].astype(jnp.float32),
                                  inv_hw, N)
                o_ref[...] = gate.astype(o_ref.dtype)

        Cr = w1.shape[0]
        out = pl.pallas_call(
            body2,
            out_shape=jax.ShapeDtypeStruct((N, Cout), x_nchw.dtype),
            grid=(nsteps,),
            in_specs=[pl.BlockSpec((ipb, C, HW), lambda k: (k, 0, 0)),
                      pl.BlockSpec((Cr, C), lambda k: (0, 0)),
                      pl.BlockSpec((Cout, Cr), lambda k: (0, 0))],
            out_specs=pl.BlockSpec((N, Cout), lambda k: (0, 0)),
            scratch_shapes=[pltpu.VMEM((nsteps, ipb, C), jnp.float32),
                            pltpu.VMEM((nsteps, ipb, C), jnp.float32)],
            compiler_params=pltpu.CompilerParams(
                dimension_semantics=("arbitrary",),
                vmem_limit_bytes=vmem_limit,
            ),
            cost_estimate=pl.CostEstimate(
                flops=2 * N * C * HW + 4 * N * (C * Cr + Cr * Cout),
                transcendentals=N * Cout,
                bytes_accessed=N * C * HW * itemsize + N * Cout * itemsize,
            ),
        )(x, w1, w2)
        return out.reshape(N, Cout, 1, 1)

    thw = max(_LANE,
              (_POOL_BLOCK_BYTES // (_round_up(C, 8) * itemsize))
              // _LANE * _LANE)
    num_k = int(pl.cdiv(HW, thw))
    needs_mask = (HW % thw) != 0
    Cr = w1.shape[0]

    def pbody(x_ref, s_ref, m_ref, s_acc, m_acc):
        k = pl.program_id(1)

        @pl.when(k == 0)
        def _init():
            s_acc[...] = jnp.zeros_like(s_acc)
            m_acc[...] = jnp.full_like(m_acc, -jnp.inf)

        xv = x_ref[0]

        def _accum(xs, xm):
            s_acc[...] += jnp.sum(xs, axis=1, keepdims=True)
            m_acc[...] = jnp.maximum(m_acc[...],
                                     jnp.max(xm, axis=1, keepdims=True))

        if needs_mask:
            @pl.when(k < num_k - 1)
            def _full():
                _accum(xv.astype(jnp.float32), xv.astype(jnp.float32))

            @pl.when(k == num_k - 1)
            def _tail():
                lane = jax.lax.broadcasted_iota(jnp.int32, (C, thw), 1)
                valid = (k * thw + lane) < HW
                _accum(jnp.where(valid, xv.astype(jnp.float32), 0.0),
                       jnp.where(valid, xv.astype(jnp.float32), -jnp.inf))
        else:
            _accum(xv.astype(jnp.float32), xv.astype(jnp.float32))

        @pl.when(k == num_k - 1)
        def _fin():
            s_ref[0] = s_acc[...]
            m_ref[0] = m_acc[...]

    psum, pmax = pl.pallas_call(
        pbody,
        out_shape=(jax.ShapeDtypeStruct((N, C, 1), jnp.float32),
                   jax.ShapeDtypeStruct((N, C, 1), jnp.float32)),
        grid=(N, num_k),
        in_specs=[pl.BlockSpec((1, C, thw), lambda n, k: (n, 0, k))],
        out_specs=(pl.BlockSpec((1, C, 1), lambda n, k: (n, 0, 0)),
                   pl.BlockSpec((1, C, 1), lambda n, k: (n, 0, 0))),
        scratch_shapes=[pltpu.VMEM((C, 1), jnp.float32),
                        pltpu.VMEM((C, 1), jnp.float32)],
        compiler_params=pltpu.CompilerParams(
            dimension_semantics=("parallel", "arbitrary"),
            vmem_limit_bytes=int(min(
                _VMEM_CAP,
                2 * _round_up(C, 8) * thw * itemsize + 8 * 1024 * 1024)),
        ),
    )(x)

    sums = psum.reshape(N, C)
    maxs = pmax.reshape(N, C)

    def mlp_body(s_ref, m_ref, w1_ref, w2_ref, o_ref):
        gate = _gate_rows(s_ref[...], m_ref[...],
                          w1_ref[...].astype(jnp.float32),
                          w2_ref[...].astype(jnp.float32), inv_hw, N)
        o_ref[...] = gate.astype(o_ref.dtype)

    out = pl.pallas_call(
        mlp_body,
        out_shape=jax.ShapeDtypeStruct((N, Cout), x_nchw.dtype),
    )(sums, maxs, w1, w2)
    return out.reshape(N, Cout, 1, 1)


# speedup vs baseline: 1.1011x; 1.1011x over previous
import jax
import jax.numpy as jnp
from jax.experimental import pallas as pl
from jax.experimental.pallas import tpu as pltpu

_LANE = 128
_POOL_BLOCK_BYTES = 16 * 1024 * 1024
_VMEM_CAP = 48 * 1024 * 1024


def _round_up(v, m):
    return -(-v // m) * m


def _gate_rows(sum_rows, max_rows, w1, w2, inv_hw, n):
    p = jnp.concatenate([sum_rows * inv_hw, max_rows], axis=0)
    h = jax.lax.dot_general(p, w1, (((1,), (1,)), ((), ())),
                            preferred_element_type=jnp.float32)
    h = jnp.maximum(h, 0.0)
    o = jax.lax.dot_general(h, w2, (((1,), (1,)), ((), ())),
                            preferred_element_type=jnp.float32)
    return jax.nn.sigmoid(o[:n, :] + o[n:, :])


def kernel(x_nchw, w1, w2):
    N, C, H, W = x_nchw.shape
    Cout = w2.shape[0]
    HW = H * W
    inv_hw = 1.0 / float(HW)
    itemsize = jnp.dtype(x_nchw.dtype).itemsize

    x = x_nchw.reshape(N, C, HW)

    ipb = 1
    for cand in (16, 8, 4, 2):
        if N % cand == 0 and cand * C * HW * itemsize <= _POOL_BLOCK_BYTES:
            ipb = cand
            break
    nsteps = N // ipb
    block_bytes = ipb * _round_up(C, 8) * _round_up(HW, _LANE) * itemsize
    vmem_limit = int(min(_VMEM_CAP, 2 * block_bytes + 8 * 1024 * 1024))

    if block_bytes <= _POOL_BLOCK_BYTES:
        def body2(x_ref, w1_ref, w2_ref, o_ref, s_rows, m_rows):
            k = pl.program_id(0)
            xv = x_ref[...]
            s_rows[k] = jnp.sum(xv, axis=2)
            m_rows[k] = jnp.max(xv, axis=2)

            @pl.when(k == nsteps - 1)
            def _fin():
                s_all = jnp.concatenate(
                    [s_rows[i] for i in range(nsteps)], axis=0)
                m_all = jnp.concatenate(
                    [m_rows[i] for i in range(nsteps)], axis=0)
                gate = _gate_rows(s_all, m_all,
                                  w1_ref[...].astype(jnp.float32),
                                  w2_ref[...].astype(jnp.float32),
                                  inv_hw, N)
                o_ref[...] = gate.astype(o_ref.dtype)

        Cr = w1.shape[0]
        out = pl.pallas_call(
            body2,
            out_shape=jax.ShapeDtypeStruct((N, Cout), x_nchw.dtype),
            grid=(nsteps,),
            in_specs=[pl.BlockSpec((ipb, C, HW), lambda k: (k, 0, 0)),
                      pl.BlockSpec((Cr, C), lambda k: (0, 0)),
                      pl.BlockSpec((Cout, Cr), lambda k: (0, 0))],
            out_specs=pl.BlockSpec((N, Cout), lambda k: (0, 0)),
            scratch_shapes=[pltpu.VMEM((nsteps, ipb, C), jnp.float32),
                            pltpu.VMEM((nsteps, ipb, C), jnp.float32)],
            compiler_params=pltpu.CompilerParams(
                dimension_semantics=("arbitrary",),
                vmem_limit_bytes=vmem_limit,
            ),
            cost_estimate=pl.CostEstimate(
                flops=2 * N * C * HW + 4 * N * (C * Cr + Cr * Cout),
                transcendentals=N * Cout,
                bytes_accessed=N * C * HW * itemsize + N * Cout * itemsize,
            ),
        )(x, w1, w2)
        return out.reshape(N, Cout, 1, 1)

    thw = max(_LANE,
              (_POOL_BLOCK_BYTES // (_round_up(C, 8) * itemsize))
              // _LANE * _LANE)
    num_k = int(pl.cdiv(HW, thw))
    needs_mask = (HW % thw) != 0
    Cr = w1.shape[0]

    def pbody(x_ref, s_ref, m_ref, s_acc, m_acc):
        k = pl.program_id(1)

        @pl.when(k == 0)
        def _init():
            s_acc[...] = jnp.zeros_like(s_acc)
            m_acc[...] = jnp.full_like(m_acc, -jnp.inf)

        xv = x_ref[0]

        def _accum(xs, xm):
            s_acc[...] += jnp.sum(xs, axis=1, keepdims=True)
            m_acc[...] = jnp.maximum(m_acc[...],
                                     jnp.max(xm, axis=1, keepdims=True))

        if needs_mask:
            @pl.when(k < num_k - 1)
            def _full():
                _accum(xv.astype(jnp.float32), xv.astype(jnp.float32))

            @pl.when(k == num_k - 1)
            def _tail():
                lane = jax.lax.broadcasted_iota(jnp.int32, (C, thw), 1)
                valid = (k * thw + lane) < HW
                _accum(jnp.where(valid, xv.astype(jnp.float32), 0.0),
                       jnp.where(valid, xv.astype(jnp.float32), -jnp.inf))
        else:
            _accum(xv.astype(jnp.float32), xv.astype(jnp.float32))

        @pl.when(k == num_k - 1)
        def _fin():
            s_ref[0] = s_acc[...]
            m_ref[0] = m_acc[...]

    psum, pmax = pl.pallas_call(
        pbody,
        out_shape=(jax.ShapeDtypeStruct((N, C, 1), jnp.float32),
                   jax.ShapeDtypeStruct((N, C, 1), jnp.float32)),
        grid=(N, num_k),
        in_specs=[pl.BlockSpec((1, C, thw), lambda n, k: (n, 0, k))],
        out_specs=(pl.BlockSpec((1, C, 1), lambda n, k: (n, 0, 0)),
                   pl.BlockSpec((1, C, 1), lambda n, k: (n, 0, 0))),
        scratch_shapes=[pltpu.VMEM((C, 1), jnp.float32),
                        pltpu.VMEM((C, 1), jnp.float32)],
        compiler_params=pltpu.CompilerParams(
            dimension_semantics=("parallel", "arbitrary"),
            vmem_limit_bytes=int(min(
                _VMEM_CAP,
                2 * _round_up(C, 8) * thw * itemsize + 8 * 1024 * 1024)),
        ),
    )(x)

    sums = psum.reshape(N, C)
    maxs = pmax.reshape(N, C)

    def mlp_body(s_ref, m_ref, w1_ref, w2_ref, o_ref):
        gate = _gate_rows(s_ref[...], m_ref[...],
                          w1_ref[...].astype(jnp.float32),
                          w2_ref[...].astype(jnp.float32), inv_hw, N)
        o_ref[...] = gate.astype(o_ref.dtype)

    out = pl.pallas_call(
        mlp_body,
        out_shape=jax.ShapeDtypeStruct((N, Cout), x_nchw.dtype),
    )(sums, maxs, w1, w2)
    return out.reshape(N, Cout, 1, 1)


# speedup vs baseline: 1.1177x; 1.0151x over previous
import jax
import jax.numpy as jnp
from jax.experimental import pallas as pl
from jax.experimental.pallas import tpu as pltpu

_LANE = 128
_POOL_BLOCK_BYTES = 16 * 1024 * 1024
_VMEM_CAP = 48 * 1024 * 1024


def _round_up(v, m):
    return -(-v // m) * m


def _gate_rows(sum_rows, max_rows, w1, w2, inv_hw, n):
    p = jnp.concatenate([sum_rows * inv_hw, max_rows], axis=0)
    h = jax.lax.dot_general(p, w1, (((1,), (1,)), ((), ())),
                            preferred_element_type=jnp.float32)
    h = jnp.maximum(h, 0.0)
    o = jax.lax.dot_general(h, w2, (((1,), (1,)), ((), ())),
                            preferred_element_type=jnp.float32)
    return jax.nn.sigmoid(o[:n, :] + o[n:, :])


def kernel(x_nchw, w1, w2):
    N, C, H, W = x_nchw.shape
    Cout = w2.shape[0]
    HW = H * W
    inv_hw = 1.0 / float(HW)
    itemsize = jnp.dtype(x_nchw.dtype).itemsize

    x = x_nchw.reshape(N, C, HW)

    ipb = 1
    for cand in (16, 8, 4, 2):
        if N % cand == 0 and cand * C * HW * itemsize <= _POOL_BLOCK_BYTES:
            ipb = cand
            break
    nsteps = N // ipb
    block_bytes = ipb * _round_up(C, 8) * _round_up(HW, _LANE) * itemsize
    vmem_limit = int(min(_VMEM_CAP, 2 * block_bytes + 8 * 1024 * 1024))

    chunk = 0
    for cand in (4, 8, 2):
        if N % cand == 0 and cand * C * HW * itemsize <= 4 * 1024 * 1024:
            chunk = cand
            break
    if chunk and N // chunk >= 2:
        nchunks = N // chunk
        depth = min(4, nchunks)
        Cr = w1.shape[0]

        def mbody(x_hbm, w1_ref, w2_ref, o_ref, bufs, sems, s_rows, m_rows):
            for c in range(depth):
                pltpu.make_async_copy(
                    x_hbm.at[pl.ds(c * chunk, chunk)],
                    bufs.at[c], sems.at[c]).start()
            for c in range(nchunks):
                slot = c % depth
                pltpu.make_async_copy(
                    bufs.at[slot], bufs.at[slot], sems.at[slot]).wait()
                xv = bufs[slot]
                s_rows[c] = jnp.sum(xv, axis=2)
                m_rows[c] = jnp.max(xv, axis=2)
                nxt = c + depth
                if nxt < nchunks:
                    pltpu.make_async_copy(
                        x_hbm.at[pl.ds(nxt * chunk, chunk)],
                        bufs.at[slot], sems.at[slot]).start()
            s_all = jnp.concatenate(
                [s_rows[i] for i in range(nchunks)], axis=0)
            m_all = jnp.concatenate(
                [m_rows[i] for i in range(nchunks)], axis=0)
            gate = _gate_rows(s_all, m_all,
                              w1_ref[...].astype(jnp.float32),
                              w2_ref[...].astype(jnp.float32),
                              inv_hw, N)
            o_ref[...] = gate.astype(o_ref.dtype)

        buf_bytes = depth * chunk * _round_up(C, 8) * HW * itemsize
        out = pl.pallas_call(
            mbody,
            out_shape=jax.ShapeDtypeStruct((N, Cout), x_nchw.dtype),
            in_specs=[pl.BlockSpec(memory_space=pl.ANY),
                      pl.BlockSpec((Cr, C), lambda: (0, 0)),
                      pl.BlockSpec((Cout, Cr), lambda: (0, 0))],
            out_specs=pl.BlockSpec((N, Cout), lambda: (0, 0)),
            scratch_shapes=[
                pltpu.VMEM((depth, chunk, C, HW), x_nchw.dtype),
                pltpu.SemaphoreType.DMA((depth,)),
                pltpu.VMEM((nchunks, chunk, C), jnp.float32),
                pltpu.VMEM((nchunks, chunk, C), jnp.float32),
            ],
            compiler_params=pltpu.CompilerParams(
                vmem_limit_bytes=int(min(_VMEM_CAP,
                                         buf_bytes + 8 * 1024 * 1024)),
            ),
            cost_estimate=pl.CostEstimate(
                flops=2 * N * C * HW + 4 * N * (C * Cr + Cr * Cout),
                transcendentals=N * Cout,
                bytes_accessed=N * C * HW * itemsize + N * Cout * itemsize,
            ),
        )(x, w1, w2)
        return out.reshape(N, Cout, 1, 1)

    if block_bytes <= _POOL_BLOCK_BYTES:
        def body2(x_ref, w1_ref, w2_ref, o_ref, s_rows, m_rows):
            k = pl.program_id(0)
            xv = x_ref[...]
            s_rows[k] = jnp.sum(xv, axis=2)
            m_rows[k] = jnp.max(xv, axis=2)

            @pl.when(k == nsteps - 1)
            def _fin():
                s_all = jnp.concatenate(
                    [s_rows[i] for i in range(nsteps)], axis=0)
                m_all = jnp.concatenate(
                    [m_rows[i] for i in range(nsteps)], axis=0)
                gate = _gate_rows(s_all, m_all,
                                  w1_ref[...].astype(jnp.float32),
                                  w2_ref[...].astype(jnp.float32),
                                  inv_hw, N)
                o_ref[...] = gate.astype(o_ref.dtype)

        Cr = w1.shape[0]
        out = pl.pallas_call(
            body2,
            out_shape=jax.ShapeDtypeStruct((N, Cout), x_nchw.dtype),
            grid=(nsteps,),
            in_specs=[pl.BlockSpec((ipb, C, HW), lambda k: (k, 0, 0)),
                      pl.BlockSpec((Cr, C), lambda k: (0, 0)),
                      pl.BlockSpec((Cout, Cr), lambda k: (0, 0))],
            out_specs=pl.BlockSpec((N, Cout), lambda k: (0, 0)),
            scratch_shapes=[pltpu.VMEM((nsteps, ipb, C), jnp.float32),
                            pltpu.VMEM((nsteps, ipb, C), jnp.float32)],
            compiler_params=pltpu.CompilerParams(
                dimension_semantics=("arbitrary",),
                vmem_limit_bytes=vmem_limit,
            ),
            cost_estimate=pl.CostEstimate(
                flops=2 * N * C * HW + 4 * N * (C * Cr + Cr * Cout),
                transcendentals=N * Cout,
                bytes_accessed=N * C * HW * itemsize + N * Cout * itemsize,
            ),
        )(x, w1, w2)
        return out.reshape(N, Cout, 1, 1)

    thw = max(_LANE,
              (_POOL_BLOCK_BYTES // (_round_up(C, 8) * itemsize))
              // _LANE * _LANE)
    num_k = int(pl.cdiv(HW, thw))
    needs_mask = (HW % thw) != 0
    Cr = w1.shape[0]

    def pbody(x_ref, s_ref, m_ref, s_acc, m_acc):
        k = pl.program_id(1)

        @pl.when(k == 0)
        def _init():
            s_acc[...] = jnp.zeros_like(s_acc)
            m_acc[...] = jnp.full_like(m_acc, -jnp.inf)

        xv = x_ref[0]

        def _accum(xs, xm):
            s_acc[...] += jnp.sum(xs, axis=1, keepdims=True)
            m_acc[...] = jnp.maximum(m_acc[...],
                                     jnp.max(xm, axis=1, keepdims=True))

        if needs_mask:
            @pl.when(k < num_k - 1)
            def _full():
                _accum(xv.astype(jnp.float32), xv.astype(jnp.float32))

            @pl.when(k == num_k - 1)
            def _tail():
                lane = jax.lax.broadcasted_iota(jnp.int32, (C, thw), 1)
                valid = (k * thw + lane) < HW
                _accum(jnp.where(valid, xv.astype(jnp.float32), 0.0),
                       jnp.where(valid, xv.astype(jnp.float32), -jnp.inf))
        else:
            _accum(xv.astype(jnp.float32), xv.astype(jnp.float32))

        @pl.when(k == num_k - 1)
        def _fin():
            s_ref[0] = s_acc[...]
            m_ref[0] = m_acc[...]

    psum, pmax = pl.pallas_call(
        pbody,
        out_shape=(jax.ShapeDtypeStruct((N, C, 1), jnp.float32),
                   jax.ShapeDtypeStruct((N, C, 1), jnp.float32)),
        grid=(N, num_k),
        in_specs=[pl.BlockSpec((1, C, thw), lambda n, k: (n, 0, k))],
        out_specs=(pl.BlockSpec((1, C, 1), lambda n, k: (n, 0, 0)),
                   pl.BlockSpec((1, C, 1), lambda n, k: (n, 0, 0))),
        scratch_shapes=[pltpu.VMEM((C, 1), jnp.float32),
                        pltpu.VMEM((C, 1), jnp.float32)],
        compiler_params=pltpu.CompilerParams(
            dimension_semantics=("parallel", "arbitrary"),
            vmem_limit_bytes=int(min(
                _VMEM_CAP,
                2 * _round_up(C, 8) * thw * itemsize + 8 * 1024 * 1024)),
        ),
    )(x)

    sums = psum.reshape(N, C)
    maxs = pmax.reshape(N, C)

    def mlp_body(s_ref, m_ref, w1_ref, w2_ref, o_ref):
        gate = _gate_rows(s_ref[...], m_ref[...],
                          w1_ref[...].astype(jnp.float32),
                          w2_ref[...].astype(jnp.float32), inv_hw, N)
        o_ref[...] = gate.astype(o_ref.dtype)

    out = pl.pallas_call(
        mlp_body,
        out_shape=jax.ShapeDtypeStruct((N, Cout), x_nchw.dtype),
    )(sums, maxs, w1, w2)
    return out.reshape(N, Cout, 1, 1)
